# Initial kernel scaffold; baseline (speedup 1.0000x reference)
import jax
import jax.numpy as jnp
from jax.experimental import pallas as pl


def kernel(atom_types, chemistry_types, emb_table, chem_table):
    raise NotImplementedError("write your pallas kernel here")


# speedup vs baseline: 5.5113x; 5.5113x over previous
import functools

import jax
import jax.numpy as jnp
from jax import lax
from jax.experimental import pallas as pl
from jax.experimental.pallas import tpu as pltpu
from jax.experimental.pallas import tpu_sc as plsc


def _sc_lookup(aidx, cidx, table, atom_subrows):
    N = aidx.shape[0]
    info = plsc.get_sparse_core_info()
    NW = info.num_cores * info.num_subcores
    PW = N // NW
    CP = 512
    PS = 32
    NST = CP // PS
    NCH = PW // CP

    mesh = plsc.VectorSubcoreMesh(core_axis_name="c", subcore_axis_name="s")

    @functools.partial(
        pl.kernel,
        out_type=jax.ShapeDtypeStruct((N * 3, 32), jnp.float32),
        mesh=mesh,
        scratch_types=[
            pltpu.VMEM((CP,), jnp.int32),
            pltpu.VMEM((CP,), jnp.int32),
            pltpu.VMEM((CP * 3,), jnp.int32),
            pltpu.VMEM((CP * 3, 32), jnp.float32),
            pltpu.SemaphoreType.DMA,
        ],
        compiler_params=pltpu.CompilerParams(
            needs_layout_passes=False, use_tc_tiling_on_sc=False),
    )
    def k(aidx_hbm, cidx_hbm, table_hbm, out_hbm, aidx_v, cidx_v, idx3_v,
          rows_v, sem):
        wid = lax.axis_index("s") * info.num_cores + lax.axis_index("c")
        base = wid * PW
        t16 = lax.broadcasted_iota(jnp.int32, (16,), 0)

        def chunk_body(i, carry):
            pbase = base + i * CP
            pltpu.sync_copy(aidx_hbm.at[pl.ds(pbase, CP)], aidx_v)
            pltpu.sync_copy(cidx_hbm.at[pl.ds(pbase, CP)], cidx_v)
            for t in range(CP // 16):
                o = t * 16
                a16 = aidx_v[pl.ds(o, 16)]
                c16 = cidx_v[pl.ds(o, 16)]
                col = o * 3 + 3 * t16
                plsc.store_scatter(idx3_v, [col], a16 * 2)
                plsc.store_scatter(idx3_v, [col + 1], a16 * 2 + 1)
                plsc.store_scatter(idx3_v, [col + 2], c16 + atom_subrows)
            copies = []
            for g in range(NST):
                copies.append(pltpu.async_copy(
                    table_hbm.at[idx3_v.at[pl.ds(g * PS * 3, PS * 3)]],
                    rows_v.at[pl.ds(g * PS * 3, PS * 3)],
                    sem,
                ))
            for cp in copies:
                cp.wait()
            pltpu.sync_copy(rows_v, out_hbm.at[pl.ds(pbase * 3, CP * 3)])
            return carry

        lax.fori_loop(0, NCH, chunk_body, 0)

    return k(aidx, cidx, table)


def kernel(atom_types, chemistry_types, emb_table, chem_table):
    B, L = atom_types.shape
    D1 = emb_table.shape[1]
    D2 = chem_table.shape[1]
    a = atom_types.reshape(-1).astype(jnp.int32)
    c = chemistry_types.reshape(-1).astype(jnp.int32)
    sub = emb_table.reshape(-1, D2)
    table = jnp.concatenate([sub, chem_table], axis=0)
    out = _sc_lookup(a, c, table, sub.shape[0])
    return out.reshape(B, L, D1 + D2)
